# baseline (speedup 1.0000x reference)
import functools

import jax
import jax.numpy as jnp
from jax import lax
from jax.experimental import pallas as pl
from jax.experimental.pallas import tpu as pltpu
from jax.experimental.pallas import tpu_sc as plsc

HIDDEN = 768

_info = plsc.get_sparse_core_info()
_NC, _NS = _info.num_cores, _info.num_subcores
_NW = _NC * _NS


def _sc_gather_slice(ids_flat, word_table, seq, pos_sl, slice_idx, chunk):
    n_tokens = ids_flat.shape[0]
    batch = n_tokens // seq
    sl_tokens = batch * pos_sl
    b_per_w = sl_tokens // _NW
    chunk = min(chunk, b_per_w)
    n_chunks = b_per_w // chunk
    w_per_batch = _NW // batch
    mesh = plsc.VectorSubcoreMesh(core_axis_name="c", subcore_axis_name="s")

    @functools.partial(
        pl.kernel,
        mesh=mesh,
        out_type=jax.ShapeDtypeStruct((sl_tokens, HIDDEN), jnp.float32),
        scratch_types=[
            pltpu.VMEM((2, chunk), jnp.int32),
            pltpu.VMEM((2, chunk, HIDDEN), jnp.float32),
            pltpu.SemaphoreType.DMA,
            pltpu.SemaphoreType.DMA,
        ],
    )
    def gather_kernel(idx_hbm, table_hbm, out_hbm, idx_v, rows_v, sem0, sem1):
        wid = lax.axis_index("s") * _NC + lax.axis_index("c")
        b = wid // w_per_batch
        j = wid % w_per_batch
        src_base = b * seq + slice_idx * pos_sl + j * b_per_w
        dst_base = wid * b_per_w
        sems = (sem0, sem1)
        copies = [None, None]
        pltpu.sync_copy(idx_hbm.at[pl.ds(src_base, chunk)], idx_v.at[0])
        copies[0] = pltpu.async_copy(table_hbm.at[idx_v.at[0]], rows_v.at[0], sems[0])
        for c in range(n_chunks):
            cur = c % 2
            nxt = (c + 1) % 2
            if c + 1 < n_chunks:
                off_n = src_base + (c + 1) * chunk
                pltpu.sync_copy(idx_hbm.at[pl.ds(off_n, chunk)], idx_v.at[nxt])
                copies[nxt] = pltpu.async_copy(
                    table_hbm.at[idx_v.at[nxt]], rows_v.at[nxt], sems[nxt])
            copies[cur].wait()
            pltpu.sync_copy(rows_v.at[cur],
                            out_hbm.at[pl.ds(dst_base + c * chunk, chunk)])

    return gather_kernel(ids_flat, word_table)


def _tc_ln_body(g_ref, tt_ref, pos_ref, type_ref, w_ref, b_ref, o_ref):
    t0 = type_ref[0:1, :]
    t1 = type_ref[1:2, :]
    tt = tt_ref[...].astype(jnp.float32)
    x = g_ref[...] + pos_ref[...] + t0 + tt * (t1 - t0)
    mu = jnp.mean(x, axis=-1, keepdims=True)
    d = x - mu
    var = jnp.mean(d * d, axis=-1, keepdims=True)
    o_ref[...] = d * lax.rsqrt(var + 1e-12) * w_ref[...] + b_ref[...]


def _tc_ln_slice(gathered_sl, tti, pos_table, type_table, ln_w, ln_b,
                 prev, n_tokens, blk, seq, pos_sl, slice_idx):
    batch = n_tokens // seq
    pb_sl = pos_sl // blk
    seq_blocks = seq // blk
    blk0 = slice_idx * pb_sl
    grid = (pb_sl, batch)
    loc = lambda p, b: (b * pb_sl + p, 0)
    glob = lambda p, b: (b * seq_blocks + blk0 + p, 0)
    specs = [
        pl.BlockSpec((blk, HIDDEN), loc),
        pl.BlockSpec((blk, 1), glob),
        pl.BlockSpec((blk, HIDDEN), lambda p, b: (blk0 + p, 0)),
        pl.BlockSpec((2, HIDDEN), lambda p, b: (0, 0)),
        pl.BlockSpec((1, HIDDEN), lambda p, b: (0, 0)),
        pl.BlockSpec((1, HIDDEN), lambda p, b: (0, 0)),
    ]
    args = [gathered_sl, tti, pos_table, type_table, ln_w, ln_b]
    aliases = {}
    if prev is not None:
        body = lambda _p, *refs: _tc_ln_body(*refs)
        specs = [pl.BlockSpec(memory_space=pl.MemorySpace.ANY)] + specs
        args = [prev] + args
        aliases = {0: 0}
    else:
        body = _tc_ln_body
    return pl.pallas_call(
        body,
        grid=grid,
        in_specs=specs,
        out_specs=pl.BlockSpec((blk, HIDDEN), glob),
        out_shape=jax.ShapeDtypeStruct((n_tokens, HIDDEN), jnp.float32),
        input_output_aliases=aliases,
    )(*args)


def kernel(input_ids, token_type_ids, word_table, pos_table, type_table, ln_w, ln_b):
    batch, seq = input_ids.shape
    n_tokens = batch * seq
    ids_flat = input_ids.reshape(-1).astype(jnp.int32)
    tti = token_type_ids.reshape(-1, 1).astype(jnp.int32)
    ln_w2 = ln_w.reshape(1, HIDDEN)
    ln_b2 = ln_b.reshape(1, HIDDEN)

    n_slices = 4
    pos_sl = seq // n_slices
    blk = 512
    gathered = [
        _sc_gather_slice(ids_flat, word_table, seq, pos_sl, i, chunk=64)
        for i in range(n_slices)
    ]
    out = None
    for i in range(n_slices):
        out = _tc_ln_slice(gathered[i], tti, pos_table, type_table,
                           ln_w2, ln_b2, out, n_tokens, blk, seq, pos_sl, i)
    return out.reshape(batch, seq, HIDDEN)


# speedup vs baseline: 1.0724x; 1.0724x over previous
import functools

import jax
import jax.numpy as jnp
from jax import lax
from jax.experimental import pallas as pl
from jax.experimental.pallas import tpu as pltpu
from jax.experimental.pallas import tpu_sc as plsc

HIDDEN = 768

_info = plsc.get_sparse_core_info()
_NC, _NS = _info.num_cores, _info.num_subcores
_NW = _NC * _NS


def _sc_gather(ids2d, word_table, chunk):
    batch, seq = ids2d.shape
    n_tokens = batch * seq
    b_per_w = n_tokens // _NW
    n_chunks = b_per_w // chunk
    w_per_batch = _NW // batch
    mesh = plsc.VectorSubcoreMesh(core_axis_name="c", subcore_axis_name="s")

    @functools.partial(
        pl.kernel,
        mesh=mesh,
        out_type=jax.ShapeDtypeStruct((n_tokens, HIDDEN), jnp.float32),
        scratch_types=[
            pltpu.VMEM((2, chunk), jnp.int32),
            pltpu.VMEM((2, chunk, HIDDEN), jnp.float32),
            pltpu.SemaphoreType.DMA,
            pltpu.SemaphoreType.DMA,
        ],
    )
    def gather_kernel(idx_hbm, table_hbm, out_hbm, idx_v, rows_v, sem0, sem1):
        wid = lax.axis_index("s") * _NC + lax.axis_index("c")
        b = wid // w_per_batch
        col = (wid % w_per_batch) * b_per_w
        dst_base = wid * b_per_w
        sems = (sem0, sem1)
        copies = [None, None]
        pltpu.sync_copy(idx_hbm.at[b, pl.ds(col, chunk)], idx_v.at[0])
        copies[0] = pltpu.async_copy(table_hbm.at[idx_v.at[0]], rows_v.at[0], sems[0])
        for c in range(n_chunks):
            cur = c % 2
            nxt = (c + 1) % 2
            if c + 1 < n_chunks:
                pltpu.sync_copy(idx_hbm.at[b, pl.ds(col + (c + 1) * chunk, chunk)],
                                idx_v.at[nxt])
                copies[nxt] = pltpu.async_copy(
                    table_hbm.at[idx_v.at[nxt]], rows_v.at[nxt], sems[nxt])
            copies[cur].wait()
            pltpu.sync_copy(rows_v.at[cur],
                            out_hbm.at[pl.ds(dst_base + c * chunk, chunk)])

    return gather_kernel(ids2d, word_table)


def _tc_ln_body(g_ref, tt_ref, pos_ref, type_ref, w_ref, b_ref, o_ref):
    t0 = type_ref[0:1, :]
    t1 = type_ref[1:2, :]
    tt = tt_ref[...].astype(jnp.float32)
    x = g_ref[...] + pos_ref[...] + t0 + tt * (t1 - t0)
    mu = jnp.mean(x, axis=-1, keepdims=True)
    d = x - mu
    var = jnp.mean(d * d, axis=-1, keepdims=True)
    o_ref[...] = d * lax.rsqrt(var + 1e-12) * w_ref[...] + b_ref[...]


def _tc_ln(gathered, tti, pos_table, type_table, ln_w, ln_b, n_tokens, blk, seq):
    pos_blocks = seq // blk
    batch = n_tokens // seq
    grid = (pos_blocks, batch)
    tok = lambda p, b: (b * pos_blocks + p, 0)
    return pl.pallas_call(
        _tc_ln_body,
        grid=grid,
        in_specs=[
            pl.BlockSpec((blk, HIDDEN), tok),
            pl.BlockSpec((blk, 1), tok),
            pl.BlockSpec((blk, HIDDEN), lambda p, b: (p, 0)),
            pl.BlockSpec((2, HIDDEN), lambda p, b: (0, 0)),
            pl.BlockSpec((1, HIDDEN), lambda p, b: (0, 0)),
            pl.BlockSpec((1, HIDDEN), lambda p, b: (0, 0)),
        ],
        out_specs=pl.BlockSpec((blk, HIDDEN), tok),
        out_shape=jax.ShapeDtypeStruct((n_tokens, HIDDEN), jnp.float32),
    )(gathered, tti, pos_table, type_table, ln_w, ln_b)


def kernel(input_ids, token_type_ids, word_table, pos_table, type_table, ln_w, ln_b):
    batch, seq = input_ids.shape
    n_tokens = batch * seq
    tti = token_type_ids.reshape(-1, 1).astype(jnp.int32)

    gathered = _sc_gather(input_ids.astype(jnp.int32), word_table, chunk=64)
    out = _tc_ln(
        gathered, tti, pos_table, type_table,
        ln_w.reshape(1, HIDDEN), ln_b.reshape(1, HIDDEN),
        n_tokens, blk=1024, seq=seq,
    )
    return out.reshape(batch, seq, HIDDEN)


# speedup vs baseline: 1.0734x; 1.0009x over previous
import functools

import jax
import jax.numpy as jnp
from jax import lax
from jax.experimental import pallas as pl
from jax.experimental.pallas import tpu as pltpu
from jax.experimental.pallas import tpu_sc as plsc

HIDDEN = 768

_info = plsc.get_sparse_core_info()
_NC, _NS = _info.num_cores, _info.num_subcores
_NW = _NC * _NS


def _sc_gather(ids2d, word_table, chunk):
    batch, seq = ids2d.shape
    n_tokens = batch * seq
    b_per_w = n_tokens // _NW
    n_chunks = b_per_w // chunk
    w_per_batch = _NW // batch
    mesh = plsc.VectorSubcoreMesh(core_axis_name="c", subcore_axis_name="s")

    @functools.partial(
        pl.kernel,
        mesh=mesh,
        out_type=jax.ShapeDtypeStruct((n_tokens, HIDDEN), jnp.float32),
        scratch_types=[
            pltpu.VMEM((2, chunk), jnp.int32),
            pltpu.VMEM((2, chunk, HIDDEN), jnp.float32),
            pltpu.SemaphoreType.DMA,
            pltpu.SemaphoreType.DMA,
        ],
    )
    def gather_kernel(idx_hbm, table_hbm, out_hbm, idx_v, rows_v, sem0, sem1):
        wid = lax.axis_index("s") * _NC + lax.axis_index("c")
        b = wid // w_per_batch
        col = (wid % w_per_batch) * b_per_w
        dst_base = wid * b_per_w
        sems = (sem0, sem1)
        pltpu.sync_copy(idx_hbm.at[b, pl.ds(col, chunk)], idx_v.at[0])
        pltpu.async_copy(table_hbm.at[idx_v.at[0]], rows_v.at[0], sems[0])

        def body(c, _):
            cur = lax.rem(c, 2)
            nxt = 1 - cur

            @pl.when(c + 1 < n_chunks)
            def _prefetch():
                pltpu.sync_copy(idx_hbm.at[b, pl.ds(col + (c + 1) * chunk, chunk)],
                                idx_v.at[nxt])

                @pl.when(nxt == 0)
                def _():
                    pltpu.async_copy(table_hbm.at[idx_v.at[0]], rows_v.at[0], sems[0])

                @pl.when(nxt == 1)
                def _():
                    pltpu.async_copy(table_hbm.at[idx_v.at[1]], rows_v.at[1], sems[1])

            @pl.when(cur == 0)
            def _():
                pltpu.make_async_copy(table_hbm.at[idx_v.at[0]], rows_v.at[0],
                                      sems[0]).wait()
                pltpu.sync_copy(rows_v.at[0],
                                out_hbm.at[pl.ds(dst_base + c * chunk, chunk)])

            @pl.when(cur == 1)
            def _():
                pltpu.make_async_copy(table_hbm.at[idx_v.at[1]], rows_v.at[1],
                                      sems[1]).wait()
                pltpu.sync_copy(rows_v.at[1],
                                out_hbm.at[pl.ds(dst_base + c * chunk, chunk)])

            return 0

        lax.fori_loop(0, n_chunks, body, 0)

    return gather_kernel(ids2d, word_table)


def _tc_ln_body(g_ref, tt_ref, pos_ref, type_ref, w_ref, b_ref, o_ref):
    t0 = type_ref[0:1, :]
    t1 = type_ref[1:2, :]
    tt = tt_ref[...].astype(jnp.float32)
    x = g_ref[...] + pos_ref[...] + t0 + tt * (t1 - t0)
    mu = jnp.mean(x, axis=-1, keepdims=True)
    d = x - mu
    var = jnp.mean(d * d, axis=-1, keepdims=True)
    o_ref[...] = d * lax.rsqrt(var + 1e-12) * w_ref[...] + b_ref[...]


def _tc_ln(gathered, tti, pos_table, type_table, ln_w, ln_b, n_tokens, blk, seq):
    pos_blocks = seq // blk
    batch = n_tokens // seq
    grid = (pos_blocks, batch)
    tok = lambda p, b: (b * pos_blocks + p, 0)
    return pl.pallas_call(
        _tc_ln_body,
        grid=grid,
        in_specs=[
            pl.BlockSpec((blk, HIDDEN), tok),
            pl.BlockSpec((blk, 1), tok),
            pl.BlockSpec((blk, HIDDEN), lambda p, b: (p, 0)),
            pl.BlockSpec((2, HIDDEN), lambda p, b: (0, 0)),
            pl.BlockSpec((1, HIDDEN), lambda p, b: (0, 0)),
            pl.BlockSpec((1, HIDDEN), lambda p, b: (0, 0)),
        ],
        out_specs=pl.BlockSpec((blk, HIDDEN), tok),
        out_shape=jax.ShapeDtypeStruct((n_tokens, HIDDEN), jnp.float32),
    )(gathered, tti, pos_table, type_table, ln_w, ln_b)


def kernel(input_ids, token_type_ids, word_table, pos_table, type_table, ln_w, ln_b):
    batch, seq = input_ids.shape
    n_tokens = batch * seq
    tti = token_type_ids.reshape(-1, 1).astype(jnp.int32)

    gathered = _sc_gather(input_ids.astype(jnp.int32), word_table, chunk=64)
    out = _tc_ln(
        gathered, tti, pos_table, type_table,
        ln_w.reshape(1, HIDDEN), ln_b.reshape(1, HIDDEN),
        n_tokens, blk=1024, seq=seq,
    )
    return out.reshape(batch, seq, HIDDEN)


# speedup vs baseline: 1.1103x; 1.0344x over previous
import functools

import jax
import jax.numpy as jnp
from jax import lax
from jax.experimental import pallas as pl
from jax.experimental.pallas import tpu as pltpu
from jax.experimental.pallas import tpu_sc as plsc

HIDDEN = 768

_info = plsc.get_sparse_core_info()
_NC, _NS = _info.num_cores, _info.num_subcores
_NW = _NC * _NS


def _sc_gather(ids2d, word_table, chunk):
    batch, seq = ids2d.shape
    n_tokens = batch * seq
    b_per_w = n_tokens // _NW
    n_chunks = b_per_w // chunk
    w_per_batch = _NW // batch
    mesh = plsc.VectorSubcoreMesh(core_axis_name="c", subcore_axis_name="s")

    @functools.partial(
        pl.kernel,
        mesh=mesh,
        out_type=jax.ShapeDtypeStruct((n_tokens, HIDDEN), jnp.float32),
        scratch_types=[
            pltpu.VMEM((2, chunk), jnp.int32),
            pltpu.VMEM((2, chunk, HIDDEN), jnp.float32),
            pltpu.SemaphoreType.DMA,
            pltpu.SemaphoreType.DMA,
        ],
    )
    def gather_kernel(idx_hbm, table_hbm, out_hbm, idx_v, rows_v, sem0, sem1):
        wid = lax.axis_index("s") * _NC + lax.axis_index("c")
        b = wid // w_per_batch
        col = (wid % w_per_batch) * b_per_w
        dst_base = wid * b_per_w
        sems = (sem0, sem1)
        pltpu.sync_copy(idx_hbm.at[b, pl.ds(col, chunk)], idx_v.at[0])
        pltpu.async_copy(table_hbm.at[idx_v.at[0]], rows_v.at[0], sems[0])

        def body(c, _):
            cur = lax.rem(c, 2)
            nxt = 1 - cur

            @pl.when(c + 1 < n_chunks)
            def _prefetch():
                pltpu.sync_copy(idx_hbm.at[b, pl.ds(col + (c + 1) * chunk, chunk)],
                                idx_v.at[nxt])

                @pl.when(nxt == 0)
                def _():
                    pltpu.async_copy(table_hbm.at[idx_v.at[0]], rows_v.at[0], sems[0])

                @pl.when(nxt == 1)
                def _():
                    pltpu.async_copy(table_hbm.at[idx_v.at[1]], rows_v.at[1], sems[1])

            @pl.when(cur == 0)
            def _():
                pltpu.make_async_copy(table_hbm.at[idx_v.at[0]], rows_v.at[0],
                                      sems[0]).wait()
                pltpu.sync_copy(rows_v.at[0],
                                out_hbm.at[pl.ds(dst_base + c * chunk, chunk)])

            @pl.when(cur == 1)
            def _():
                pltpu.make_async_copy(table_hbm.at[idx_v.at[1]], rows_v.at[1],
                                      sems[1]).wait()
                pltpu.sync_copy(rows_v.at[1],
                                out_hbm.at[pl.ds(dst_base + c * chunk, chunk)])

            return 0

        lax.fori_loop(0, n_chunks, body, 0)

    return gather_kernel(ids2d, word_table)


def _tc_ln_body(g_ref, tt_ref, pos_ref, type_ref, w_ref, b_ref, o_ref):
    t0 = type_ref[0:1, :]
    t1 = type_ref[1:2, :]
    tt = tt_ref[...].astype(jnp.float32)
    x = g_ref[...] + pos_ref[...] + t0 + tt * (t1 - t0)
    mu = jnp.mean(x, axis=-1, keepdims=True)
    d = x - mu
    var = jnp.mean(d * d, axis=-1, keepdims=True)
    o_ref[...] = d * lax.rsqrt(var + 1e-12) * w_ref[...] + b_ref[...]


def _tc_ln(gathered, tti, pos_table, type_table, ln_w, ln_b, n_tokens, blk, seq):
    pos_blocks = seq // blk
    batch = n_tokens // seq
    grid = (pos_blocks, batch)
    tok = lambda p, b: (b * pos_blocks + p, 0)
    return pl.pallas_call(
        _tc_ln_body,
        grid=grid,
        in_specs=[
            pl.BlockSpec((blk, HIDDEN), tok),
            pl.BlockSpec((blk, 1), tok),
            pl.BlockSpec((blk, HIDDEN), lambda p, b: (p, 0)),
            pl.BlockSpec((2, HIDDEN), lambda p, b: (0, 0)),
            pl.BlockSpec((1, HIDDEN), lambda p, b: (0, 0)),
            pl.BlockSpec((1, HIDDEN), lambda p, b: (0, 0)),
        ],
        out_specs=pl.BlockSpec((blk, HIDDEN), tok),
        out_shape=jax.ShapeDtypeStruct((n_tokens, HIDDEN), jnp.float32),
    )(gathered, tti, pos_table, type_table, ln_w, ln_b)


def kernel(input_ids, token_type_ids, word_table, pos_table, type_table, ln_w, ln_b):
    batch, seq = input_ids.shape
    n_tokens = batch * seq
    tti = token_type_ids.reshape(-1, 1).astype(jnp.int32)

    gathered = _sc_gather(input_ids.astype(jnp.int32), word_table, chunk=64)
    out = _tc_ln(
        gathered, tti, pos_table, type_table,
        ln_w.reshape(1, HIDDEN), ln_b.reshape(1, HIDDEN),
        n_tokens, blk=2048, seq=seq,
    )
    return out.reshape(batch, seq, HIDDEN)
